# baseline (speedup 1.0000x reference)
import functools

import jax
import jax.numpy as jnp
import numpy as np
from jax.experimental import pallas as pl
from jax.experimental.pallas import tpu as pltpu

N = 1024
F = 200
B = 8
T = 35
O = 40
KB = 8

_BN_RSQRT = 1.0 / np.sqrt(1.0 + 1e-5)


def _elu(v):
    return jnp.where(v > 0, v, jnp.exp(v) - 1.0)


def _gat_kernel(x_ref, wt_ref, as_ref, ad_ref, o_ref):
    x0 = x_ref[0:N, :]
    h = jax.lax.dot_general(x0, wt_ref[...], (((1,), (1,)), ((), ())),
                            preferred_element_type=jnp.float32)
    es = jnp.sum(h * as_ref[...], axis=1, keepdims=True)
    ed = jnp.sum(h * ad_ref[...], axis=1, keepdims=True)
    ones = jnp.ones_like(es)
    lhs = jnp.concatenate([ed, ones], axis=1)
    rhs = jnp.concatenate([ones, es], axis=1)
    epre = jax.lax.dot_general(lhs, rhs, (((1,), (1,)), ((), ())),
                               preferred_element_type=jnp.float32)
    e = jnp.where(epre > 0, epre, 0.2 * epre)
    jj = jax.lax.broadcasted_iota(jnp.int32, (N, N), 0)
    ii = jax.lax.broadcasted_iota(jnp.int32, (N, N), 1)
    em = jnp.where(jj != ii, e, -1e30)
    m = jnp.max(em, axis=1, keepdims=True)
    p = jnp.exp(em - m)
    s = jnp.sum(p, axis=1, keepdims=True)
    alpha = p / (s + 1e-16)
    g = jax.lax.dot_general(alpha, h, (((1,), (0,)), ((), ())),
                            preferred_element_type=jnp.float32)
    o_ref[0:N, :] = x_ref[0:N, :] + g
    o_ref[N:, :] = x_ref[N:, :]


def _conv1_kernel(x_ref, w_ref, s_ref, b_ref, o_ref):
    y = jax.lax.dot_general(x_ref[...], w_ref[...], (((1,), (0,)), ((), ())),
                            preferred_element_type=jnp.float32)
    o_ref[...] = _elu(y * s_ref[...] + b_ref[...])


def _conv2_kernel(p_ref, w_ref, s2_ref, b2_ref, pj_ref, pb_ref, o_ref,
                  acc_ref):
    k = pl.program_id(0)
    part = jax.lax.dot_general(p_ref[...], w_ref[...],
                               (((1,), (0,)), ((), ())),
                               preferred_element_type=jnp.float32)

    @pl.when(k == 0)
    def _():
        acc_ref[...] = part

    @pl.when(k > 0)
    def _():
        acc_ref[...] = acc_ref[...] + part

    @pl.when(k == KB - 1)
    def _():
        c2 = _elu(acc_ref[...] * s2_ref[...] + b2_ref[...])
        c3 = jax.lax.dot_general(c2, pj_ref[...], (((1,), (0,)), ((), ())),
                                 preferred_element_type=jnp.float32)
        o_ref[...] = c3 + pb_ref[...]


def _mlp_kernel(e_ref, w1_ref, b1_ref, w2_ref, b2_ref, g_ref, be_ref, o_ref):
    y = jax.lax.dot_general(e_ref[...], w1_ref[...], (((1,), (1,)), ((), ())),
                            preferred_element_type=jnp.float32) + b1_ref[...]
    z = 0.5 * y * (1.0 + jax.lax.erf(y * np.float32(1.0 / np.sqrt(2.0))))
    z2 = jax.lax.dot_general(z, w2_ref[...], (((1,), (1,)), ((), ())),
                             preferred_element_type=jnp.float32) + b2_ref[...]
    yo = y + z2
    mu = jnp.mean(yo, axis=1, keepdims=True)
    d = yo - mu
    var = jnp.mean(d * d, axis=1, keepdims=True)
    o_ref[...] = d * jax.lax.rsqrt(var + 1e-5) * g_ref[...] + be_ref[...]


def _full(shape):
    return pl.BlockSpec(shape, lambda *_: tuple(0 for _ in shape))


def kernel(x, W_gat, att_src, att_dst, b_gat, conv1_w, conv1_b, bn1_g, bn1_b,
           conv2_w, conv2_b, bn2_g, bn2_b, proj_w, proj_b, W1, b1, W2, b2,
           ln_g, ln_b, edge_index):
    del edge_index
    f32 = jnp.float32
    x2 = x.reshape(B * N, F)

    w26 = conv1_w[:, 0, 0, :]
    w_eff = sum(jnp.pad(w26, ((0, 0), (j, 4 - j))) for j in range(5)) / 5.0
    rel = jnp.arange(F)[:, None] - 5 * jnp.arange(T)[None, :]
    valid = (rel >= 0) & (rel < 30)
    W3 = jnp.where(valid[:, :, None], w_eff.T[jnp.clip(rel, 0, 29)], 0.0)
    W3f = W3.reshape(F, T * O)
    scale1 = (bn1_g * _BN_RSQRT).reshape(1, O)
    bias_t = ((conv1_b[None, :] + (b_gat @ W3f).reshape(T, O))
              * scale1 + bn1_b[None, :])
    scale1_col = jnp.tile(scale1, (1, T))
    bias_col = bias_t.reshape(1, T * O)
    W2flat = jnp.transpose(conv2_w[:, :, :, 0], (2, 1, 0)).reshape(N * O, O)
    scale2 = (bn2_g * _BN_RSQRT).reshape(1, O)
    bias2 = (conv2_b * scale2[0] + bn2_b).reshape(1, O)
    projT = proj_w[:, :, 0, 0].T
    pb = proj_b.reshape(1, O)

    x1 = pl.pallas_call(
        _gat_kernel,
        out_shape=jax.ShapeDtypeStruct((B * N, F), f32),
        in_specs=[_full((B * N, F)), _full((F, F)),
                  _full((1, F)), _full((1, F))],
        out_specs=_full((B * N, F)),
    )(x2, W_gat, att_src.reshape(1, F), att_dst.reshape(1, F))

    MB = 8
    MBLK = B * N // MB
    P4 = pl.pallas_call(
        _conv1_kernel,
        grid=(MB,),
        out_shape=jax.ShapeDtypeStruct((B * N, T * O), f32),
        in_specs=[
            pl.BlockSpec((MBLK, F), lambda m: (m, 0)),
            pl.BlockSpec((F, T * O), lambda m: (0, 0)),
            pl.BlockSpec((1, T * O), lambda m: (0, 0)),
            pl.BlockSpec((1, T * O), lambda m: (0, 0)),
        ],
        out_specs=pl.BlockSpec((MBLK, T * O), lambda m: (m, 0)),
    )(x1, W3f, scale1_col, bias_col)

    P3v = (P4.reshape(B, N, T, O).transpose(0, 2, 1, 3)
           .reshape(B * T, N * O))
    KBLK = N * O // KB
    c3 = pl.pallas_call(
        _conv2_kernel,
        grid=(KB,),
        out_shape=jax.ShapeDtypeStruct((B * T, O), f32),
        in_specs=[
            pl.BlockSpec((B * T, KBLK), lambda k: (0, k)),
            pl.BlockSpec((KBLK, O), lambda k: (k, 0)),
            pl.BlockSpec((1, O), lambda k: (0, 0)),
            pl.BlockSpec((1, O), lambda k: (0, 0)),
            pl.BlockSpec((O, O), lambda k: (0, 0)),
            pl.BlockSpec((1, O), lambda k: (0, 0)),
        ],
        out_specs=pl.BlockSpec((B * T, O), lambda k: (0, 0)),
        scratch_shapes=[pltpu.VMEM((B * T, O), f32)],
    )(P3v, W2flat, scale2, bias2, projT, pb)

    emb = c3.reshape(B, T * O)
    out = pl.pallas_call(
        _mlp_kernel,
        out_shape=jax.ShapeDtypeStruct((B, N), f32),
        in_specs=[_full((B, T * O)), _full((N, T * O)), _full((1, N)),
                  _full((N, N)), _full((1, N)), _full((1, N)), _full((1, N))],
        out_specs=_full((B, N)),
    )(emb, W1, b1.reshape(1, N), W2, b2.reshape(1, N),
      ln_g.reshape(1, N), ln_b.reshape(1, N))
    return out


# speedup vs baseline: 1.3338x; 1.3338x over previous
import functools

import jax
import jax.numpy as jnp
import numpy as np
from jax.experimental import pallas as pl
from jax.experimental.pallas import tpu as pltpu

N = 1024
F = 200
B = 8
T = 35
O = 40
KB = 8

_BN_RSQRT = 1.0 / np.sqrt(1.0 + 1e-5)


def _elu(v):
    return jnp.where(v > 0, v, jnp.exp(v) - 1.0)


def _gat_kernel(x_ref, wt_ref, as_ref, ad_ref, o_ref):
    x0 = x_ref[0:N, :]
    h = jax.lax.dot_general(x0, wt_ref[...], (((1,), (1,)), ((), ())),
                            preferred_element_type=jnp.float32)
    es = jnp.sum(h * as_ref[...], axis=1, keepdims=True)
    ed = jnp.sum(h * ad_ref[...], axis=1, keepdims=True)
    ones = jnp.ones_like(es)
    lhs = jnp.concatenate([ed, ones], axis=1)
    rhs = jnp.concatenate([ones, es], axis=1)
    epre = jax.lax.dot_general(lhs, rhs, (((1,), (1,)), ((), ())),
                               preferred_element_type=jnp.float32)
    e = jnp.where(epre > 0, epre, 0.2 * epre)
    jj = jax.lax.broadcasted_iota(jnp.int32, (N, N), 0)
    ii = jax.lax.broadcasted_iota(jnp.int32, (N, N), 1)
    em = jnp.where(jj != ii, e, -1e30)
    m = jnp.max(em, axis=1, keepdims=True)
    p = jnp.exp(em - m)
    s = jnp.sum(p, axis=1, keepdims=True)
    alpha = p / (s + 1e-16)
    g = jax.lax.dot_general(alpha, h, (((1,), (0,)), ((), ())),
                            preferred_element_type=jnp.float32)
    o_ref[0:N, :] = (x_ref[0:N, :] + g).astype(jnp.bfloat16)
    o_ref[N:, :] = x_ref[N:, :].astype(jnp.bfloat16)


def _conv1_kernel(x_ref, w_ref, s_ref, b_ref, o_ref):
    t = pl.program_id(0)
    y = jax.lax.dot_general(x_ref[...], w_ref[0], (((1,), (0,)), ((), ())),
                            preferred_element_type=jnp.float32)
    yb = y * s_ref[...] + b_ref[pl.ds(t, 1), :]
    o_ref[...] = _elu(yb).reshape(B, 1, N, O)


def _conv2_kernel(p_ref, w_ref, s2_ref, b2_ref, pj_ref, pb_ref, o_ref,
                  acc_ref):
    k = pl.program_id(0)
    part = jax.lax.dot_general(p_ref[...], w_ref[...],
                               (((1,), (0,)), ((), ())),
                               preferred_element_type=jnp.float32)

    @pl.when(k == 0)
    def _():
        acc_ref[...] = part

    @pl.when(k > 0)
    def _():
        acc_ref[...] = acc_ref[...] + part

    @pl.when(k == KB - 1)
    def _():
        c2 = _elu(acc_ref[...] * s2_ref[...] + b2_ref[...])
        c3 = jax.lax.dot_general(c2, pj_ref[...], (((1,), (0,)), ((), ())),
                                 preferred_element_type=jnp.float32)
        o_ref[...] = c3 + pb_ref[...]


def _mlp_kernel(e_ref, w1_ref, b1_ref, w2_ref, b2_ref, g_ref, be_ref, o_ref):
    y = jax.lax.dot_general(e_ref[...], w1_ref[...], (((1,), (1,)), ((), ())),
                            preferred_element_type=jnp.float32) + b1_ref[...]
    z = 0.5 * y * (1.0 + jax.lax.erf(y * np.float32(1.0 / np.sqrt(2.0))))
    z2 = jax.lax.dot_general(z, w2_ref[...], (((1,), (1,)), ((), ())),
                             preferred_element_type=jnp.float32) + b2_ref[...]
    yo = y + z2
    mu = jnp.mean(yo, axis=1, keepdims=True)
    d = yo - mu
    var = jnp.mean(d * d, axis=1, keepdims=True)
    o_ref[...] = d * jax.lax.rsqrt(var + 1e-5) * g_ref[...] + be_ref[...]


def _full(shape):
    return pl.BlockSpec(shape, lambda *_: tuple(0 for _ in shape))


def kernel(x, W_gat, att_src, att_dst, b_gat, conv1_w, conv1_b, bn1_g, bn1_b,
           conv2_w, conv2_b, bn2_g, bn2_b, proj_w, proj_b, W1, b1, W2, b2,
           ln_g, ln_b, edge_index):
    del edge_index
    f32 = jnp.float32
    x2 = x.reshape(B * N, F)

    w26 = conv1_w[:, 0, 0, :]
    w_eff = sum(jnp.pad(w26, ((0, 0), (j, 4 - j))) for j in range(5)) / 5.0
    rel = jnp.arange(F)[:, None] - 5 * jnp.arange(T)[None, :]
    valid = (rel >= 0) & (rel < 30)
    W3 = jnp.where(valid[:, :, None], w_eff.T[jnp.clip(rel, 0, 29)], 0.0)
    W3f = W3.reshape(F, T * O)
    W3t = jnp.transpose(W3, (1, 0, 2)).astype(jnp.bfloat16)
    scale1 = (bn1_g * _BN_RSQRT).reshape(1, O)
    bias_t = ((conv1_b[None, :] + (b_gat @ W3f).reshape(T, O))
              * scale1 + bn1_b[None, :])
    scale1_col = jnp.tile(scale1, (1, T))
    bias_col = bias_t.reshape(1, T * O)
    W2flat = jnp.transpose(conv2_w[:, :, :, 0], (2, 1, 0)).reshape(N * O, O)
    scale2 = (bn2_g * _BN_RSQRT).reshape(1, O)
    bias2 = (conv2_b * scale2[0] + bn2_b).reshape(1, O)
    projT = proj_w[:, :, 0, 0].T
    pb = proj_b.reshape(1, O)

    x1 = pl.pallas_call(
        _gat_kernel,
        out_shape=jax.ShapeDtypeStruct((B * N, F), jnp.bfloat16),
        in_specs=[_full((B * N, F)), _full((F, F)),
                  _full((1, F)), _full((1, F))],
        out_specs=_full((B * N, F)),
    )(x2, W_gat, att_src.reshape(1, F), att_dst.reshape(1, F))

    P3 = pl.pallas_call(
        _conv1_kernel,
        grid=(T,),
        out_shape=jax.ShapeDtypeStruct((B, T, N, O), f32),
        in_specs=[
            pl.BlockSpec((B * N, F), lambda t: (0, 0)),
            pl.BlockSpec((1, F, O), lambda t: (t, 0, 0)),
            pl.BlockSpec((1, O), lambda t: (0, 0)),
            pl.BlockSpec((T, O), lambda t: (0, 0)),
        ],
        out_specs=pl.BlockSpec((B, 1, N, O), lambda t: (0, t, 0, 0)),
    )(x1, W3t, scale1, bias_t)

    P3v = P3.reshape(B * T, N * O)
    KBLK = N * O // KB
    c3 = pl.pallas_call(
        _conv2_kernel,
        grid=(KB,),
        out_shape=jax.ShapeDtypeStruct((B * T, O), f32),
        in_specs=[
            pl.BlockSpec((B * T, KBLK), lambda k: (0, k)),
            pl.BlockSpec((KBLK, O), lambda k: (k, 0)),
            pl.BlockSpec((1, O), lambda k: (0, 0)),
            pl.BlockSpec((1, O), lambda k: (0, 0)),
            pl.BlockSpec((O, O), lambda k: (0, 0)),
            pl.BlockSpec((1, O), lambda k: (0, 0)),
        ],
        out_specs=pl.BlockSpec((B * T, O), lambda k: (0, 0)),
        scratch_shapes=[pltpu.VMEM((B * T, O), f32)],
    )(P3v, W2flat, scale2, bias2, projT, pb)

    emb = c3.reshape(B, T * O)
    out = pl.pallas_call(
        _mlp_kernel,
        out_shape=jax.ShapeDtypeStruct((B, N), f32),
        in_specs=[_full((B, T * O)), _full((N, T * O)), _full((1, N)),
                  _full((N, N)), _full((1, N)), _full((1, N)), _full((1, N))],
        out_specs=_full((B, N)),
    )(emb, W1, b1.reshape(1, N), W2, b2.reshape(1, N),
      ln_g.reshape(1, N), ln_b.reshape(1, N))
    return out
